# baseline (speedup 1.0000x reference)
import jax
import jax.numpy as jnp
from jax import lax
from jax.experimental import pallas as pl
from jax.experimental.pallas import tpu as pltpu
from jax.experimental.pallas import tpu_sc as plsc

G = 4
N = 512
H = 32
S = 512

NC = 2
NS = 16
NW = NC * NS

ROWS = G * N
RPW = ROWS // NW
CHUNK = 2
LANES = 16
NCHUNK = RPW // CHUNK
HP = H // 2

TB = 512


def _pack2d(e):
    u = jax.lax.bitcast_convert_type(e, jnp.uint32)
    r = (u + jnp.uint32(0x7FFF) + ((u >> 16) & jnp.uint32(1))) >> 16
    w = r[:, :HP] | (r[:, HP:] << 16)
    return jax.lax.bitcast_convert_type(w.T, jnp.int32)


def _prep_body(sp_ref, enc_ref, encr_ref, spt_ref, penc_ref, pencr_ref):
    spt_ref[0] = sp_ref[0].T
    penc_ref[...] = _pack2d(enc_ref[...])
    pencr_ref[...] = _pack2d(encr_ref[...])


def _tc_prep(sp, enc, encr):
    return pl.pallas_call(
        _prep_body,
        out_shape=(
            jax.ShapeDtypeStruct((G, N, N), jnp.int32),
            jax.ShapeDtypeStruct((HP, S), jnp.int32),
            jax.ShapeDtypeStruct((HP, S), jnp.int32),
        ),
        grid=(G, N // TB, N // TB),
        in_specs=[
            pl.BlockSpec((1, TB, TB), lambda g, a, b: (g, b, a)),
            pl.BlockSpec((S, H), lambda g, a, b: (0, 0)),
            pl.BlockSpec((S, H), lambda g, a, b: (0, 0)),
        ],
        out_specs=(
            pl.BlockSpec((1, TB, TB), lambda g, a, b: (g, a, b)),
            pl.BlockSpec((HP, S), lambda g, a, b: (0, 0)),
            pl.BlockSpec((HP, S), lambda g, a, b: (0, 0)),
        ),
    )(sp, enc, encr)


def _sc_body(ab_hbm, sp_hbm, spt_hbm, encp_hbm, encrp_hbm, out_hbm,
             encp_v, encrp_v, sp_v, spt_v, ab_v, obuf_v, sem_in, sem_out):
    cid = lax.axis_index("c")
    sid = lax.axis_index("s")
    wid = sid * NC + cid

    tsem = sem_in.at[0]
    for k in range(HP):
        pltpu.async_copy(encp_hbm.at[k], encp_v.at[pl.ds(k * S, S)], tsem)
        pltpu.async_copy(encrp_hbm.at[k], encrp_v.at[pl.ds(k * S, S)], tsem)
    for k in range(HP):
        pltpu.make_async_copy(encp_hbm.at[k], encp_v.at[pl.ds(k * S, S)],
                              tsem).wait()
        pltpu.make_async_copy(encrp_hbm.at[k], encrp_v.at[pl.ds(k * S, S)],
                              tsem).wait()

    row0 = wid * RPW
    g = row0 // N
    i_base = row0 % N

    def istart(ck, par):
        i0 = i_base + ck * CHUNK
        pltpu.async_copy(sp_hbm.at[g, pl.ds(i0, CHUNK), :], sp_v.at[par],
                         sem_in.at[par])
        pltpu.async_copy(spt_hbm.at[g, pl.ds(i0, CHUNK), :], spt_v.at[par],
                         sem_in.at[par])
        pltpu.async_copy(ab_hbm.at[g, pl.ds(i0, CHUNK), :], ab_v.at[par],
                         sem_in.at[par])

    def iwait(ck, par):
        i0 = i_base + ck * CHUNK
        pltpu.make_async_copy(sp_hbm.at[g, pl.ds(i0, CHUNK), :],
                              sp_v.at[par], sem_in.at[par]).wait()
        pltpu.make_async_copy(spt_hbm.at[g, pl.ds(i0, CHUNK), :],
                              spt_v.at[par], sem_in.at[par]).wait()
        pltpu.make_async_copy(ab_hbm.at[g, pl.ds(i0, CHUNK), :],
                              ab_v.at[par], sem_in.at[par]).wait()

    def ostart(ck, par):
        i0 = i_base + ck * CHUNK
        pltpu.async_copy(obuf_v.at[par],
                         out_hbm.at[g, :, pl.ds(i0, CHUNK), :],
                         sem_out.at[par])

    def owait(ck, par):
        i0 = i_base + ck * CHUNK
        pltpu.make_async_copy(obuf_v.at[par],
                              out_hbm.at[g, :, pl.ds(i0, CHUNK), :],
                              sem_out.at[par]).wait()

    istart(0, 0)
    istart(1, 1)

    @pl.loop(0, NCHUNK, step=2)
    def _chunk(ck0):
        for par in range(2):
            ck = ck0 + par
            iwait(ck, par)

            @pl.when(ck >= 2)
            def _():
                owait(ck - 2, par)

            GRP = 8

            for c in range(CHUNK):
                @plsc.parallel_loop(0, N // LANES, unroll=1)
                def _t(t):
                    sl = pl.ds(t * LANES, LANES)
                    spvec = sp_v[par, c, sl]
                    sptvec = spt_v[par, c, sl]
                    abvec = ab_v[par, c, sl]

                    def gathers(k0):
                        ks = range(k0, k0 + GRP)
                        fwds = [plsc.load_gather(encp_v.at[pl.ds(k * S, S)],
                                                 [spvec]) for k in ks]
                        revs = [plsc.load_gather(encrp_v.at[pl.ds(k * S, S)],
                                                 [sptvec]) for k in ks]
                        return fwds, revs

                    def arith(k0, fwds, revs):
                        for u, k in enumerate(range(k0, k0 + GRP)):
                            pair = (plsc.bitcast(fwds[u], jnp.bfloat16)
                                    + plsc.bitcast(revs[u], jnp.bfloat16))
                            lo, hi = plsc.unpack(
                                pair, format=plsc.PackFormat.INTERLEAVED)
                            obuf_v[par, k, c, sl] = lo + abvec
                            obuf_v[par, k + HP, c, sl] = hi + abvec

                    pend = gathers(0)
                    for k0 in range(GRP, HP, GRP):
                        cur = gathers(k0)
                        arith(k0 - GRP, *pend)
                        pend = cur
                    arith(HP - GRP, *pend)

            ostart(ck, par)

            @pl.when(ck + 2 < NCHUNK)
            def _():
                istart(ck + 2, par)

    owait(NCHUNK - 2, 0)
    owait(NCHUNK - 1, 1)


@jax.jit
def kernel(attn_bias, spatial_pos, spatial_pos_encoder, spatial_pos_encoder_rev):
    spt, encp, encrp = _tc_prep(spatial_pos, spatial_pos_encoder,
                                spatial_pos_encoder_rev)

    mesh = plsc.VectorSubcoreMesh(core_axis_name="c", subcore_axis_name="s")
    run = pl.kernel(
        _sc_body,
        out_type=jax.ShapeDtypeStruct((G, H, N, N), jnp.float32),
        mesh=mesh,
        compiler_params=pltpu.CompilerParams(needs_layout_passes=False),
        scratch_types=[
            pltpu.VMEM((HP * S,), jnp.int32),
            pltpu.VMEM((HP * S,), jnp.int32),
            pltpu.VMEM((2, CHUNK, N), jnp.int32),
            pltpu.VMEM((2, CHUNK, N), jnp.int32),
            pltpu.VMEM((2, CHUNK, N), jnp.float32),
            pltpu.VMEM((2, H, CHUNK, N), jnp.float32),
            pltpu.SemaphoreType.DMA((2,)),
            pltpu.SemaphoreType.DMA((2,)),
        ],
    )
    return run(attn_bias, spatial_pos, spt, encp, encrp)


# speedup vs baseline: 1.1767x; 1.1767x over previous
import jax
import jax.numpy as jnp
from jax import lax
from jax.experimental import pallas as pl
from jax.experimental.pallas import tpu as pltpu
from jax.experimental.pallas import tpu_sc as plsc

G = 4
N = 512
H = 32
S = 512

NC = 2
NS = 16
NW = NC * NS

ROWS = G * N
RPW = ROWS // NW
CHUNK = 2
LANES = 16
NCHUNK = RPW // CHUNK
HP = H // 2

TB = 512


def _pack2d(e):
    u = jax.lax.bitcast_convert_type(e, jnp.uint32)
    r = (u + jnp.uint32(0x7FFF) + ((u >> 16) & jnp.uint32(1))) >> 16
    w = r[:, :HP] | (r[:, HP:] << 16)
    return jax.lax.bitcast_convert_type(w.T, jnp.int32)


def _prep_body(sp_ref, enc_ref, encr_ref, spt_ref, penc_ref, pencr_ref):
    spt_ref[0] = sp_ref[0].T
    penc_ref[...] = _pack2d(enc_ref[...])
    pencr_ref[...] = _pack2d(encr_ref[...])


def _tc_prep(sp, enc, encr):
    return pl.pallas_call(
        _prep_body,
        out_shape=(
            jax.ShapeDtypeStruct((G, N, N), jnp.int32),
            jax.ShapeDtypeStruct((HP, S), jnp.int32),
            jax.ShapeDtypeStruct((HP, S), jnp.int32),
        ),
        grid=(G, N // TB, N // TB),
        in_specs=[
            pl.BlockSpec((1, TB, TB), lambda g, a, b: (g, b, a)),
            pl.BlockSpec((S, H), lambda g, a, b: (0, 0)),
            pl.BlockSpec((S, H), lambda g, a, b: (0, 0)),
        ],
        out_specs=(
            pl.BlockSpec((1, TB, TB), lambda g, a, b: (g, a, b)),
            pl.BlockSpec((HP, S), lambda g, a, b: (0, 0)),
            pl.BlockSpec((HP, S), lambda g, a, b: (0, 0)),
        ),
    )(sp, enc, encr)


def _sc_body(ab_hbm, sp_hbm, spt_hbm, encp_hbm, encrp_hbm, out_hbm,
             encp_v, encrp_v, sp_v, spt_v, ab_v, obuf_v, sem_in, sem_out):
    cid = lax.axis_index("c")
    sid = lax.axis_index("s")
    wid = sid * NC + cid

    tsem = sem_in.at[0]
    for k in range(HP):
        pltpu.async_copy(encp_hbm.at[k], encp_v.at[pl.ds(k * S, S)], tsem)
        pltpu.async_copy(encrp_hbm.at[k], encrp_v.at[pl.ds(k * S, S)], tsem)
    for k in range(HP):
        pltpu.make_async_copy(encp_hbm.at[k], encp_v.at[pl.ds(k * S, S)],
                              tsem).wait()
        pltpu.make_async_copy(encrp_hbm.at[k], encrp_v.at[pl.ds(k * S, S)],
                              tsem).wait()

    row0 = wid * RPW
    g = row0 // N
    i_base = row0 % N

    def istart(ck, par):
        i0 = i_base + ck * CHUNK
        pltpu.async_copy(sp_hbm.at[g, pl.ds(i0, CHUNK), :], sp_v.at[par],
                         sem_in.at[par])
        pltpu.async_copy(spt_hbm.at[g, pl.ds(i0, CHUNK), :], spt_v.at[par],
                         sem_in.at[par])
        pltpu.async_copy(ab_hbm.at[g, pl.ds(i0, CHUNK), :], ab_v.at[par],
                         sem_in.at[par])

    def iwait(ck, par):
        i0 = i_base + ck * CHUNK
        pltpu.make_async_copy(sp_hbm.at[g, pl.ds(i0, CHUNK), :],
                              sp_v.at[par], sem_in.at[par]).wait()
        pltpu.make_async_copy(spt_hbm.at[g, pl.ds(i0, CHUNK), :],
                              spt_v.at[par], sem_in.at[par]).wait()
        pltpu.make_async_copy(ab_hbm.at[g, pl.ds(i0, CHUNK), :],
                              ab_v.at[par], sem_in.at[par]).wait()

    def ostart(ck, par):
        i0 = i_base + ck * CHUNK
        pltpu.async_copy(obuf_v.at[par],
                         out_hbm.at[g, :, pl.ds(i0, CHUNK), :],
                         sem_out.at[par])

    def owait(ck, par):
        i0 = i_base + ck * CHUNK
        pltpu.make_async_copy(obuf_v.at[par],
                              out_hbm.at[g, :, pl.ds(i0, CHUNK), :],
                              sem_out.at[par]).wait()

    istart(0, 0)
    istart(1, 1)

    @pl.loop(0, NCHUNK, step=2)
    def _chunk(ck0):
        for par in range(2):
            ck = ck0 + par
            iwait(ck, par)

            @pl.when(ck >= 2)
            def _():
                owait(ck - 2, par)

            GRP = 2

            for c in range(CHUNK):
                @plsc.parallel_loop(0, N // LANES, unroll=1)
                def _t(t):
                    sl = pl.ds(t * LANES, LANES)
                    spvec = sp_v[par, c, sl]
                    sptvec = spt_v[par, c, sl]
                    abvec = ab_v[par, c, sl]

                    def gathers(k0):
                        ks = range(k0, k0 + GRP)
                        fwds = [plsc.load_gather(encp_v.at[pl.ds(k * S, S)],
                                                 [spvec]) for k in ks]
                        revs = [plsc.load_gather(encrp_v.at[pl.ds(k * S, S)],
                                                 [sptvec]) for k in ks]
                        return fwds, revs

                    def arith(k0, fwds, revs):
                        for u, k in enumerate(range(k0, k0 + GRP)):
                            pair = (plsc.bitcast(fwds[u], jnp.bfloat16)
                                    + plsc.bitcast(revs[u], jnp.bfloat16))
                            lo, hi = plsc.unpack(
                                pair, format=plsc.PackFormat.INTERLEAVED)
                            obuf_v[par, k, c, sl] = lo + abvec
                            obuf_v[par, k + HP, c, sl] = hi + abvec

                    pend = gathers(0)
                    for k0 in range(GRP, HP, GRP):
                        cur = gathers(k0)
                        arith(k0 - GRP, *pend)
                        pend = cur
                    arith(HP - GRP, *pend)

            ostart(ck, par)

            @pl.when(ck + 2 < NCHUNK)
            def _():
                istart(ck + 2, par)

    owait(NCHUNK - 2, 0)
    owait(NCHUNK - 1, 1)


@jax.jit
def kernel(attn_bias, spatial_pos, spatial_pos_encoder, spatial_pos_encoder_rev):
    spt, encp, encrp = _tc_prep(spatial_pos, spatial_pos_encoder,
                                spatial_pos_encoder_rev)

    mesh = plsc.VectorSubcoreMesh(core_axis_name="c", subcore_axis_name="s")
    run = pl.kernel(
        _sc_body,
        out_type=jax.ShapeDtypeStruct((G, H, N, N), jnp.float32),
        mesh=mesh,
        compiler_params=pltpu.CompilerParams(needs_layout_passes=False),
        scratch_types=[
            pltpu.VMEM((HP * S,), jnp.int32),
            pltpu.VMEM((HP * S,), jnp.int32),
            pltpu.VMEM((2, CHUNK, N), jnp.int32),
            pltpu.VMEM((2, CHUNK, N), jnp.int32),
            pltpu.VMEM((2, CHUNK, N), jnp.float32),
            pltpu.VMEM((2, H, CHUNK, N), jnp.float32),
            pltpu.SemaphoreType.DMA((2,)),
            pltpu.SemaphoreType.DMA((2,)),
        ],
    )
    return run(attn_bias, spatial_pos, spt, encp, encrp)
